# baseline (speedup 1.0000x reference)
import functools

import jax
import jax.numpy as jnp
from jax import lax
from jax.experimental import pallas as pl
from jax.experimental.pallas import tpu as pltpu
from jax.experimental.pallas import tpu_sc as plsc

N_CAND = 3_200_000
N_SEG = 4096
N_SAMP = 400_000
NC = 2
NS = 16
NW = NC * NS
L = 16

C1 = 2000
K1_CHUNKS = N_CAND // (NW * C1)
C2 = 2000
K2_NCHUNK = N_SAMP // C2
K2_MAXPER = (K2_NCHUNK + NW - 1) // NW

_mesh = plsc.VectorSubcoreMesh(core_axis_name="c", subcore_axis_name="s")
_params = pltpu.CompilerParams(needs_layout_passes=False)


def _iota16():
    return lax.broadcasted_iota(jnp.int32, (L,), 0)


@functools.partial(
    pl.kernel,
    out_type=jax.ShapeDtypeStruct((NC, N_SEG), jnp.float32),
    mesh=_mesh,
    compiler_params=_params,
    scratch_types=[
        pltpu.VMEM((C1,), jnp.int32),
        pltpu.VMEM((C1,), jnp.int32),
        pltpu.VMEM((C1,), jnp.int32),
        pltpu.VMEM((C1,), jnp.int32),
        pltpu.VMEM((C1,), jnp.int32),
        pltpu.VMEM((C1,), jnp.int32),
        pltpu.VMEM((C1,), jnp.float32),
        pltpu.VMEM((C1,), jnp.float32),
        pltpu.VMEM((C1,), jnp.float32),
        pltpu.VMEM((C1,), jnp.float32),
        pltpu.VMEM((C1,), jnp.float32),
        pltpu.VMEM((C1,), jnp.float32),
        pltpu.VMEM((N_SEG,), jnp.float32),
        pltpu.VMEM((NS, N_SEG // NS), jnp.float32),
        pltpu.VMEM((N_SEG // NS,), jnp.float32),
        pltpu.VMEM_SHARED((NS * N_SEG,), jnp.float32),
        pltpu.SemaphoreType.DMA,
        pltpu.SemaphoreType.DMA,
        pltpu.SemaphoreType.DMA,
        pltpu.SemaphoreType.DMA,
        pltpu.SemaphoreType.DMA,
        pltpu.SemaphoreType.DMA,
    ],
)
def _k1(segcol, eidcol, u_hbm, pp_hbm, part_out,
        sg0, sg1, sg2, ei0, ei1, ei2, uu0, uu1, uu2, pp0, pp1, pp2,
        s_loc, vbuf, sbuf, acc_sh,
        semL0, semL1, semL2, semP0, semP1, semP2):
    cid = lax.axis_index("c")
    sid = lax.axis_index("s")
    wid = sid * NC + cid

    def zz(j, carry):
        s_loc[pl.ds(j * L, L)] = jnp.zeros((L,), jnp.float32)
        return carry

    lax.fori_loop(0, N_SEG // L, zz, 0)

    tile_base = wid * (N_CAND // NW)
    segb = [sg0, sg1, sg2]
    eidb = [ei0, ei1, ei2]
    ub = [uu0, uu1, uu2]
    pb = [pp0, pp1, pp2]
    semL = [semL0, semL1, semL2]
    semP = [semP0, semP1, semP2]

    def fire_lin(ci, b):
        base = tile_base + ci * C1
        pltpu.async_copy(segcol.at[pl.ds(base, C1)], segb[b], semL[b])
        pltpu.async_copy(eidcol.at[pl.ds(base, C1)], eidb[b], semL[b])
        pltpu.async_copy(u_hbm.at[pl.ds(base, C1)], ub[b], semL[b])

    def wait_lin_fire_p(ci, b):
        base = tile_base + ci * C1
        pltpu.make_async_copy(segcol.at[pl.ds(base, C1)], segb[b], semL[b]).wait()
        pltpu.make_async_copy(eidcol.at[pl.ds(base, C1)], eidb[b], semL[b]).wait()
        pltpu.make_async_copy(u_hbm.at[pl.ds(base, C1)], ub[b], semL[b]).wait()
        pltpu.async_copy(pp_hbm.at[eidb[b]], pb[b], semP[b])

    def wait_p_accum(b):
        pltpu.make_async_copy(pp_hbm.at[eidb[b]], pb[b], semP[b]).wait()
        seg_v, u_v, p_v = segb[b], ub[b], pb[b]

        def body(i, carry):
            for jj in range(25):
                s = pl.ds((i * 25 + jj) * L, L)
                e = jnp.exp(p_v[s] + u_v[s])
                plsc.addupdate_scatter(s_loc, [seg_v[s]], e)
            return carry

        lax.fori_loop(0, C1 // L // 25, body, 0)

    fire_lin(0, 0)
    fire_lin(1, 1)
    wait_lin_fire_p(0, 0)

    def outer(k, carry):
        c = 3 * k
        fire_lin(c + 2, 2)
        wait_lin_fire_p(c + 1, 1)
        wait_p_accum(0)
        fire_lin(c + 3, 0)
        wait_lin_fire_p(c + 2, 2)
        wait_p_accum(1)
        fire_lin(c + 4, 1)
        wait_lin_fire_p(c + 3, 0)
        wait_p_accum(2)
        return carry

    lax.fori_loop(0, (K1_CHUNKS - 2) // 3, outer, 0)
    wait_p_accum(0)
    wait_lin_fire_p(K1_CHUNKS - 1, 1)
    wait_p_accum(1)
    pltpu.sync_copy(s_loc, acc_sh.at[pl.ds(sid * N_SEG, N_SEG)])
    plsc.subcore_barrier()

    W = N_SEG // NS
    for r in range(NS):
        pltpu.sync_copy(acc_sh.at[pl.ds(r * N_SEG + sid * W, W)], vbuf.at[r])

    def red(j, carry):
        s = pl.ds(j * L, L)
        acc = vbuf[0, s]
        for r in range(1, NS):
            acc = acc + vbuf[r, s]
        sbuf[s] = acc
        return carry

    lax.fori_loop(0, W // L, red, 0)
    pltpu.sync_copy(sbuf, part_out.at[cid, pl.ds(sid * W, W)])


@functools.partial(
    pl.kernel,
    out_type=jax.ShapeDtypeStruct((N_SAMP,), jnp.float32),
    mesh=_mesh,
    compiler_params=_params,
    scratch_types=[
        pltpu.VMEM((C2,), jnp.int32),
        pltpu.VMEM((C2,), jnp.int32),
        pltpu.VMEM((C2,), jnp.int32),
        pltpu.VMEM((C2,), jnp.int32),
        pltpu.VMEM((C2,), jnp.int32),
        pltpu.VMEM((C2,), jnp.int32),
        pltpu.VMEM((C2,), jnp.int32),
        pltpu.VMEM((C2,), jnp.int32),
        pltpu.VMEM((C2,), jnp.int32),
        pltpu.VMEM((C2,), jnp.float32),
        pltpu.VMEM((C2,), jnp.float32),
        pltpu.VMEM((C2,), jnp.float32),
        pltpu.VMEM((C2,), jnp.float32),
        pltpu.VMEM((C2,), jnp.float32),
        pltpu.VMEM((C2,), jnp.float32),
        pltpu.VMEM((C2,), jnp.float32),
        pltpu.VMEM((N_SEG,), jnp.float32),
        pltpu.VMEM((N_SEG,), jnp.float32),
        pltpu.VMEM((N_SEG,), jnp.float32),
        pltpu.SemaphoreType.DMA,
        pltpu.SemaphoreType.DMA,
        pltpu.SemaphoreType.DMA,
        pltpu.SemaphoreType.DMA,
        pltpu.SemaphoreType.DMA,
        pltpu.SemaphoreType.DMA,
        pltpu.SemaphoreType.DMA,
        pltpu.SemaphoreType.DMA,
        pltpu.SemaphoreType.DMA,
        pltpu.SemaphoreType.DMA,
        pltpu.SemaphoreType.DMA,
        pltpu.SemaphoreType.DMA,
    ],
)
def _k2(ca_hbm, segcol, eidcol, u_hbm, pp_hbm, part, out_hbm,
        ca0, ca1, ca2, sg0, sg1, sg2, ei0, ei1, ei2,
        uu0, uu1, uu2, pp0, pp1, pp2,
        o_v, S_v, t0, t1,
        semC0, semC1, semC2, semG0, semG1, semG2,
        semE0, semE1, semE2, semP0, semP1, semP2):
    cid = lax.axis_index("c")
    sid = lax.axis_index("s")
    wid = sid * NC + cid

    pltpu.sync_copy(part.at[0], t0)
    pltpu.sync_copy(part.at[1], t1)

    def red(j, carry):
        s = pl.ds(j * L, L)
        S_v[s] = t0[s] + t1[s]
        return carry

    lax.fori_loop(0, N_SEG // L, red, 0)

    cab = [ca0, ca1, ca2]
    segb = [sg0, sg1, sg2]
    eidb = [ei0, ei1, ei2]
    ub = [uu0, uu1, uu2]
    pb = [pp0, pp1, pp2]
    semC = [semC0, semC1, semC2]
    semG = [semG0, semG1, semG2]
    semE = [semE0, semE1, semE2]
    semP = [semP0, semP1, semP2]

    def guard(k, fn):
        c = wid + k * NW

        @pl.when(c < K2_NCHUNK)
        def _():
            fn(c)

    def s0(k):
        b = k % 3

        def f(c):
            pltpu.async_copy(ca_hbm.at[pl.ds(c * C2, C2)], cab[b], semC[b])

        guard(k, f)

    def s1(k):
        b = k % 3

        def f(c):
            pltpu.make_async_copy(
                ca_hbm.at[pl.ds(c * C2, C2)], cab[b], semC[b]).wait()
            pltpu.async_copy(segcol.at[cab[b]], segb[b], semG[b])
            pltpu.async_copy(u_hbm.at[cab[b]], ub[b], semG[b])
            pltpu.async_copy(eidcol.at[cab[b]], eidb[b], semE[b])

        guard(k, f)

    def s2(k):
        b = k % 3

        def f(c):
            pltpu.make_async_copy(eidcol.at[cab[b]], eidb[b], semE[b]).wait()
            pltpu.async_copy(pp_hbm.at[eidb[b]], pb[b], semP[b])

        guard(k, f)

    def s3(k):
        b = k % 3

        def f(c):
            pltpu.make_async_copy(segcol.at[cab[b]], segb[b], semG[b]).wait()
            pltpu.make_async_copy(u_hbm.at[cab[b]], ub[b], semG[b]).wait()
            pltpu.make_async_copy(pp_hbm.at[eidb[b]], pb[b], semP[b]).wait()
            seg_v, u_v, p_v = segb[b], ub[b], pb[b]

            def comp(j, c2):
                for jj in range(25):
                    s = pl.ds((j * 25 + jj) * L, L)
                    Ss = plsc.load_gather(S_v, [seg_v[s]])
                    ys = jnp.exp(p_v[s] + u_v[s]) / Ss
                    o_v[s] = (1.0 - ys) + ys
                return c2

            lax.fori_loop(0, C2 // L // 25, comp, 0)
            pltpu.sync_copy(o_v, out_hbm.at[pl.ds(c * C2, C2)])

        guard(k, f)

    for k in range(K2_MAXPER + 3):
        if k >= 3:
            s3(k - 3)
        if 2 <= k < K2_MAXPER + 2:
            s2(k - 2)
        if 1 <= k < K2_MAXPER + 1:
            s1(k - 1)
        if k < K2_MAXPER:
            s0(k)


def kernel(candidate_edges, loglog_u, sampled_edges, prob_params):
    segcol = candidate_edges[:, 0]
    eidcol = candidate_edges[:, 1]
    ca = sampled_edges[:, 5]
    part = _k1(segcol, eidcol, loglog_u, prob_params)
    return _k2(ca, segcol, eidcol, loglog_u, prob_params, part)


# speedup vs baseline: 1.0443x; 1.0443x over previous
import functools

import jax
import jax.numpy as jnp
from jax import lax
from jax.experimental import pallas as pl
from jax.experimental.pallas import tpu as pltpu
from jax.experimental.pallas import tpu_sc as plsc

N_CAND = 3_200_000
N_SEG = 4096
N_SAMP = 400_000
NC = 2
NS = 16
NW = NC * NS
L = 16

C1 = 2000
K1_CHUNKS = N_CAND // (NW * C1)
C2 = 2000
K2_NCHUNK = N_SAMP // C2
K2_MAXPER = (K2_NCHUNK + NW - 1) // NW

_mesh = plsc.VectorSubcoreMesh(core_axis_name="c", subcore_axis_name="s")
_params = pltpu.CompilerParams(needs_layout_passes=False)


def _iota16():
    return lax.broadcasted_iota(jnp.int32, (L,), 0)


@functools.partial(
    pl.kernel,
    out_type=(
        jax.ShapeDtypeStruct((NC, N_SEG), jnp.float32),
        jax.ShapeDtypeStruct((N_CAND,), jnp.float32),
    ),
    mesh=_mesh,
    compiler_params=_params,
    scratch_types=[
        pltpu.VMEM((C1,), jnp.int32),
        pltpu.VMEM((C1,), jnp.int32),
        pltpu.VMEM((C1,), jnp.int32),
        pltpu.VMEM((C1,), jnp.int32),
        pltpu.VMEM((C1,), jnp.int32),
        pltpu.VMEM((C1,), jnp.int32),
        pltpu.VMEM((C1,), jnp.float32),
        pltpu.VMEM((C1,), jnp.float32),
        pltpu.VMEM((C1,), jnp.float32),
        pltpu.VMEM((C1,), jnp.float32),
        pltpu.VMEM((C1,), jnp.float32),
        pltpu.VMEM((C1,), jnp.float32),
        pltpu.VMEM((N_SEG,), jnp.float32),
        pltpu.VMEM((NS, N_SEG // NS), jnp.float32),
        pltpu.VMEM((N_SEG // NS,), jnp.float32),
        pltpu.VMEM_SHARED((NS * N_SEG,), jnp.float32),
        pltpu.SemaphoreType.DMA,
        pltpu.SemaphoreType.DMA,
        pltpu.SemaphoreType.DMA,
        pltpu.SemaphoreType.DMA,
        pltpu.SemaphoreType.DMA,
        pltpu.SemaphoreType.DMA,
        pltpu.SemaphoreType.DMA,
        pltpu.SemaphoreType.DMA,
        pltpu.SemaphoreType.DMA,
    ],
)
def _k1(segcol, eidcol, u_hbm, pp_hbm, part_out, ecol_out,
        sg0, sg1, sg2, ei0, ei1, ei2, uu0, uu1, uu2, pp0, pp1, pp2,
        s_loc, vbuf, sbuf, acc_sh,
        semL0, semL1, semL2, semP0, semP1, semP2, semW0, semW1, semW2):
    cid = lax.axis_index("c")
    sid = lax.axis_index("s")
    wid = sid * NC + cid

    def zz(j, carry):
        s_loc[pl.ds(j * L, L)] = jnp.zeros((L,), jnp.float32)
        return carry

    lax.fori_loop(0, N_SEG // L, zz, 0)

    tile_base = wid * (N_CAND // NW)
    segb = [sg0, sg1, sg2]
    eidb = [ei0, ei1, ei2]
    ub = [uu0, uu1, uu2]
    pb = [pp0, pp1, pp2]
    semL = [semL0, semL1, semL2]
    semP = [semP0, semP1, semP2]
    semW = [semW0, semW1, semW2]

    def fire_lin(ci, b):
        base = tile_base + ci * C1
        pltpu.async_copy(segcol.at[pl.ds(base, C1)], segb[b], semL[b])
        pltpu.async_copy(eidcol.at[pl.ds(base, C1)], eidb[b], semL[b])
        pltpu.async_copy(u_hbm.at[pl.ds(base, C1)], ub[b], semL[b])

    def wait_lin_fire_p(ci, b, guard_w=True):
        base = tile_base + ci * C1
        pltpu.make_async_copy(segcol.at[pl.ds(base, C1)], segb[b], semL[b]).wait()
        pltpu.make_async_copy(eidcol.at[pl.ds(base, C1)], eidb[b], semL[b]).wait()
        pltpu.make_async_copy(u_hbm.at[pl.ds(base, C1)], ub[b], semL[b]).wait()
        if guard_w:
            @pl.when(ci >= 3)
            def _():
                pltpu.make_async_copy(
                    pb[b], ecol_out.at[pl.ds(base, C1)], semW[b]).wait()

        pltpu.async_copy(pp_hbm.at[eidb[b]], pb[b], semP[b])

    def wait_p_accum(ci, b):
        base = tile_base + ci * C1
        pltpu.make_async_copy(pp_hbm.at[eidb[b]], pb[b], semP[b]).wait()
        seg_v, u_v, p_v = segb[b], ub[b], pb[b]

        def body(i, carry):
            for jj in range(25):
                s = pl.ds((i * 25 + jj) * L, L)
                e = jnp.exp(p_v[s] + u_v[s])
                plsc.addupdate_scatter(s_loc, [seg_v[s]], e)
                p_v[s] = e
            return carry

        lax.fori_loop(0, C1 // L // 25, body, 0)
        pltpu.async_copy(pb[b], ecol_out.at[pl.ds(base, C1)], semW[b])

    fire_lin(0, 0)
    fire_lin(1, 1)
    wait_lin_fire_p(0, 0, guard_w=False)

    def outer(k, carry):
        c = 3 * k
        fire_lin(c + 2, 2)
        wait_lin_fire_p(c + 1, 1)
        wait_p_accum(c, 0)
        fire_lin(c + 3, 0)
        wait_lin_fire_p(c + 2, 2)
        wait_p_accum(c + 1, 1)
        fire_lin(c + 4, 1)
        wait_lin_fire_p(c + 3, 0)
        wait_p_accum(c + 2, 2)
        return carry

    lax.fori_loop(0, (K1_CHUNKS - 2) // 3, outer, 0)
    wait_p_accum(K1_CHUNKS - 2, 0)
    wait_lin_fire_p(K1_CHUNKS - 1, 1)
    wait_p_accum(K1_CHUNKS - 1, 1)
    for b in range(3):
        ci = K1_CHUNKS - 3 + b
        bb = ci % 3
        pltpu.make_async_copy(
            pb[bb], ecol_out.at[pl.ds(tile_base + ci * C1, C1)], semW[bb]).wait()
    pltpu.sync_copy(s_loc, acc_sh.at[pl.ds(sid * N_SEG, N_SEG)])
    plsc.subcore_barrier()

    W = N_SEG // NS
    for r in range(NS):
        pltpu.sync_copy(acc_sh.at[pl.ds(r * N_SEG + sid * W, W)], vbuf.at[r])

    def red(j, carry):
        s = pl.ds(j * L, L)
        acc = vbuf[0, s]
        for r in range(1, NS):
            acc = acc + vbuf[r, s]
        sbuf[s] = acc
        return carry

    lax.fori_loop(0, W // L, red, 0)
    pltpu.sync_copy(sbuf, part_out.at[cid, pl.ds(sid * W, W)])


@functools.partial(
    pl.kernel,
    out_type=jax.ShapeDtypeStruct((N_SAMP,), jnp.float32),
    mesh=_mesh,
    compiler_params=_params,
    scratch_types=[
        pltpu.VMEM((C2,), jnp.int32),
        pltpu.VMEM((C2,), jnp.int32),
        pltpu.VMEM((C2,), jnp.int32),
        pltpu.VMEM((C2,), jnp.int32),
        pltpu.VMEM((C2,), jnp.int32),
        pltpu.VMEM((C2,), jnp.int32),
        pltpu.VMEM((C2,), jnp.float32),
        pltpu.VMEM((C2,), jnp.float32),
        pltpu.VMEM((C2,), jnp.float32),
        pltpu.VMEM((C2,), jnp.float32),
        pltpu.VMEM((N_SEG,), jnp.float32),
        pltpu.VMEM((N_SEG,), jnp.float32),
        pltpu.VMEM((N_SEG,), jnp.float32),
        pltpu.SemaphoreType.DMA,
        pltpu.SemaphoreType.DMA,
        pltpu.SemaphoreType.DMA,
        pltpu.SemaphoreType.DMA,
        pltpu.SemaphoreType.DMA,
        pltpu.SemaphoreType.DMA,
    ],
)
def _k2(ca_hbm, segcol, ecol, part, out_hbm,
        ca0, ca1, ca2, sg0, sg1, sg2, ee0, ee1, ee2,
        o_v, S_v, t0, t1,
        semC0, semC1, semC2, semG0, semG1, semG2):
    cid = lax.axis_index("c")
    sid = lax.axis_index("s")
    wid = sid * NC + cid

    pltpu.sync_copy(part.at[0], t0)
    pltpu.sync_copy(part.at[1], t1)

    def red(j, carry):
        s = pl.ds(j * L, L)
        S_v[s] = t0[s] + t1[s]
        return carry

    lax.fori_loop(0, N_SEG // L, red, 0)

    cab = [ca0, ca1, ca2]
    segb = [sg0, sg1, sg2]
    eb = [ee0, ee1, ee2]
    semC = [semC0, semC1, semC2]
    semG = [semG0, semG1, semG2]

    def guard(k, fn):
        c = wid + k * NW

        @pl.when(c < K2_NCHUNK)
        def _():
            fn(c)

    def s0(k):
        b = k % 3

        def f(c):
            pltpu.async_copy(ca_hbm.at[pl.ds(c * C2, C2)], cab[b], semC[b])

        guard(k, f)

    def s1(k):
        b = k % 3

        def f(c):
            pltpu.make_async_copy(
                ca_hbm.at[pl.ds(c * C2, C2)], cab[b], semC[b]).wait()
            pltpu.async_copy(segcol.at[cab[b]], segb[b], semG[b])
            pltpu.async_copy(ecol.at[cab[b]], eb[b], semG[b])

        guard(k, f)

    def s2(k):
        b = k % 3

        def f(c):
            pltpu.make_async_copy(segcol.at[cab[b]], segb[b], semG[b]).wait()
            pltpu.make_async_copy(ecol.at[cab[b]], eb[b], semG[b]).wait()
            seg_v, e_v = segb[b], eb[b]

            def comp(j, c2):
                for jj in range(25):
                    s = pl.ds((j * 25 + jj) * L, L)
                    Ss = plsc.load_gather(S_v, [seg_v[s]])
                    ys = e_v[s] / Ss
                    o_v[s] = (1.0 - ys) + ys
                return c2

            lax.fori_loop(0, C2 // L // 25, comp, 0)
            pltpu.sync_copy(o_v, out_hbm.at[pl.ds(c * C2, C2)])

        guard(k, f)

    for k in range(K2_MAXPER + 2):
        if k >= 2:
            s2(k - 2)
        if 1 <= k < K2_MAXPER + 1:
            s1(k - 1)
        if k < K2_MAXPER:
            s0(k)


def kernel(candidate_edges, loglog_u, sampled_edges, prob_params):
    segcol = candidate_edges[:, 0]
    eidcol = candidate_edges[:, 1]
    ca = sampled_edges[:, 5]
    part, ecol = _k1(segcol, eidcol, loglog_u, prob_params)
    return _k2(ca, segcol, ecol, part)
